# Initial kernel scaffold; baseline (speedup 1.0000x reference)
import jax
import jax.numpy as jnp
from jax.experimental import pallas as pl


def kernel(mask, R, T, focal_length, principal_point):
    raise NotImplementedError("write your pallas kernel here")


# speedup vs baseline: 1.0578x; 1.0578x over previous
import functools

import jax
import jax.numpy as jnp
import numpy as np
from jax.experimental import pallas as pl
from jax.experimental.pallas import tpu as pltpu

B = 64
H = 512
W = 512
HW = H * W
N_RAYS = 1024
N_PTS = 64
MIN_DEPTH = 0.1
MAX_DEPTH = 8.0

_K_IDX = (507451445, 1853169794)
_K_STRAT = (1948878966, 4237131848)

_TINY = float(np.finfo(np.float32).tiny)

_NT = 2048
_NCHUNK = HW // _NT
_NGROUP = N_RAYS // 8


def _u32(x):
    return jnp.uint32(x)


def _threefry(x0, x1, k0, k1):
    ks0 = np.uint32(k0)
    ks1 = np.uint32(k1)
    ks2 = np.uint32(int(ks0) ^ int(ks1) ^ 0x1BD11BDA)
    ks = (ks0, ks1, ks2)
    rots = ((13, 15, 26, 6), (17, 29, 16, 24))
    for d in range(5):
        for r in rots[d % 2]:
            x0 = x0 + x1
            x1 = (x1 << _u32(r)) | (x1 >> _u32(32 - r))
            x1 = x1 ^ x0
        x0 = x0 + ks[(d + 1) % 3]
        x1 = x1 + np.uint32((int(ks[(d + 2) % 3]) + d + 1) & 0xFFFFFFFF)
    return x0, x1


def _bits_to_unit(bits):
    fb = (bits >> _u32(9)) | _u32(0x3F800000)
    return jax.lax.bitcast_convert_type(fb, jnp.float32) - jnp.float32(1.0)


def _race_body(mask_ref, idx_ref, invp_ref):
    b = pl.program_id(0)
    invp_ref[...] = jnp.float32(1.0) / jnp.maximum(mask_ref[0], jnp.float32(1e-12))

    x0_init = (jnp.uint32(b) >> _u32(4)) + _u32(_K_IDX[0])
    lo_base = ((jnp.uint32(b) & _u32(15)) * _u32(N_RAYS)) << _u32(18)

    tile_iota = (
        jax.lax.broadcasted_iota(jnp.uint32, (8, _NT), 0) << _u32(18)
    ) | jax.lax.broadcasted_iota(jnp.uint32, (8, _NT), 1)
    lane_i32 = jax.lax.broadcasted_iota(jnp.int32, (8, _NT), 1)

    def group_body(g, _):
        ray0 = g * 8
        grp_base = lo_base + (jnp.uint32(ray0) << _u32(18)) + _u32(_K_IDX[1])

        def chunk_body(c, carry):
            best, bidx = carry
            x1 = tile_iota + (grp_base + jnp.uint32(c) * _u32(_NT))
            o0, o1 = _threefry(x0_init, x1, _K_IDX[0], _K_IDX[1])
            u = jnp.maximum(_bits_to_unit(o0 ^ o1), jnp.float32(_TINY))
            invp = invp_ref[c, :]
            score = jnp.log(u) * jnp.broadcast_to(invp[None, :], (8, _NT))
            cur = lane_i32 + c * _NT
            take = score > best
            return (jnp.where(take, score, best), jnp.where(take, cur, bidx))

        init = (jnp.full((8, _NT), -jnp.inf, jnp.float32),
                jnp.zeros((8, _NT), jnp.int32))
        best, bidx = jax.lax.fori_loop(0, _NCHUNK, chunk_body, init)
        mx = jnp.max(best, axis=1, keepdims=True)
        win = jnp.min(jnp.where(best == mx, bidx, jnp.int32(2**30)), axis=1)
        idx_ref[0, g, :] = win
        return 0

    jax.lax.fori_loop(0, _NGROUP, group_body, 0)


def _post_body(idx_ref, r9_ref, t_ref, fl_ref, pp_ref, xys_ref, dir_ref, org_ref):
    idx = idx_ref[0, 0, :]
    wcol = idx & jnp.int32(W - 1)
    hrow = idx >> jnp.int32(9)
    c0 = jnp.float32(1.0 - 1.0 / W)
    step = jnp.float32(1.0 / 256.0)
    xf = c0 - wcol.astype(jnp.float32) * step
    yf = c0 - hrow.astype(jnp.float32) * step

    a, bb, c = r9_ref[0, 0, 0], r9_ref[0, 0, 1], r9_ref[0, 0, 2]
    d, e, f = r9_ref[0, 0, 3], r9_ref[0, 0, 4], r9_ref[0, 0, 5]
    g, h, i = r9_ref[0, 0, 6], r9_ref[0, 0, 7], r9_ref[0, 0, 8]
    det = a * (e * i - f * h) - bb * (d * i - f * g) + c * (d * h - e * g)
    inv_det = jnp.float32(1.0) / det
    i00 = (e * i - f * h) * inv_det
    i01 = (c * h - bb * i) * inv_det
    i02 = (bb * f - c * e) * inv_det
    i10 = (f * g - d * i) * inv_det
    i11 = (a * i - c * g) * inv_det
    i12 = (c * d - a * f) * inv_det
    i20 = (d * h - e * g) * inv_det
    i21 = (bb * g - a * h) * inv_det
    i22 = (a * e - bb * d) * inv_det

    px, py = pp_ref[0, 0, 0], pp_ref[0, 0, 1]
    fx, fy = fl_ref[0, 0, 0], fl_ref[0, 0, 1]
    dx = (xf - px) / fx
    dy = (yf - py) / fy

    d0 = dx * i00 + dy * i10 + i20
    d1 = dx * i01 + dy * i11 + i21
    d2 = dx * i02 + dy * i12 + i22
    nrm = jnp.sqrt(d0 * d0 + d1 * d1 + d2 * d2)
    dir_ref[0, 0, 0, :] = d0 / nrm
    dir_ref[0, 1, 0, :] = d1 / nrm
    dir_ref[0, 2, 0, :] = d2 / nrm

    t0, t1, t2 = t_ref[0, 0, 0], t_ref[0, 0, 1], t_ref[0, 0, 2]
    cc0 = -(t0 * i00 + t1 * i10 + t2 * i20)
    cc1 = -(t0 * i01 + t1 * i11 + t2 * i21)
    cc2 = -(t0 * i02 + t1 * i12 + t2 * i22)
    org_ref[0, 0, 0, :] = jnp.broadcast_to(cc0, (N_RAYS,))
    org_ref[0, 1, 0, :] = jnp.broadcast_to(cc1, (N_RAYS,))
    org_ref[0, 2, 0, :] = jnp.broadcast_to(cc2, (N_RAYS,))

    xys_ref[0, 0, 0, :] = xf
    xys_ref[0, 1, 0, :] = yf


def _lengths_body(base_ref, out_ref):
    b = pl.program_id(0)
    e = (
        jax.lax.broadcasted_iota(jnp.uint32, (N_RAYS, N_PTS), 0) * _u32(N_PTS)
        + jax.lax.broadcasted_iota(jnp.uint32, (N_RAYS, N_PTS), 1)
        + jnp.uint32(b) * _u32(N_RAYS * N_PTS)
        + _u32(_K_STRAT[1])
    )
    o0, o1 = _threefry(_u32(_K_STRAT[0]), e, _K_STRAT[0], _K_STRAT[1])
    u = _bits_to_unit(o0 ^ o1)
    delta = np.float32((MAX_DEPTH - MIN_DEPTH) / (N_PTS - 1))
    jit = (u - jnp.float32(0.5)) * delta
    out_ref[0] = jnp.broadcast_to(base_ref[0][None, :], (N_RAYS, N_PTS)) + jit


@functools.partial(jax.jit)
def kernel(mask, R, T, focal_length, principal_point):
    mask3 = mask.reshape(B, _NCHUNK, _NT)

    idx3 = pl.pallas_call(
        _race_body,
        grid=(B,),
        in_specs=[pl.BlockSpec((1, _NCHUNK, _NT), lambda b: (b, 0, 0))],
        out_specs=pl.BlockSpec((1, _NGROUP, 8), lambda b: (b, 0, 0)),
        out_shape=jax.ShapeDtypeStruct((B, _NGROUP, 8), jnp.int32),
        scratch_shapes=[pltpu.VMEM((_NCHUNK, _NT), jnp.float32)],
        compiler_params=pltpu.CompilerParams(
            dimension_semantics=("arbitrary",),
        ),
    )(mask3)

    idxf = idx3.reshape(B, 1, N_RAYS)
    r9 = R.reshape(B, 1, 9)

    smem = functools.partial(pl.BlockSpec, memory_space=pltpu.SMEM)
    xys4, dir4, org4 = pl.pallas_call(
        _post_body,
        grid=(B,),
        in_specs=[
            pl.BlockSpec((1, 1, N_RAYS), lambda b: (b, 0, 0)),
            smem((1, 1, 9), lambda b: (b, 0, 0)),
            smem((1, 1, 3), lambda b: (b, 0, 0)),
            smem((1, 1, 2), lambda b: (b, 0, 0)),
            smem((1, 1, 2), lambda b: (b, 0, 0)),
        ],
        out_specs=[
            pl.BlockSpec((1, 2, 1, N_RAYS), lambda b: (b, 0, 0, 0)),
            pl.BlockSpec((1, 3, 1, N_RAYS), lambda b: (b, 0, 0, 0)),
            pl.BlockSpec((1, 3, 1, N_RAYS), lambda b: (b, 0, 0, 0)),
        ],
        out_shape=[
            jax.ShapeDtypeStruct((B, 2, 1, N_RAYS), jnp.float32),
            jax.ShapeDtypeStruct((B, 3, 1, N_RAYS), jnp.float32),
            jax.ShapeDtypeStruct((B, 3, 1, N_RAYS), jnp.float32),
        ],
    )(idxf, r9, T.reshape(B, 1, 3), focal_length.reshape(B, 1, 2),
      principal_point.reshape(B, 1, 2))

    base = jnp.linspace(MIN_DEPTH, MAX_DEPTH, N_PTS, dtype=jnp.float32)
    lengths = pl.pallas_call(
        _lengths_body,
        grid=(B,),
        in_specs=[pl.BlockSpec((1, N_PTS), lambda b: (0, 0))],
        out_specs=pl.BlockSpec((1, N_RAYS, N_PTS), lambda b: (b, 0, 0)),
        out_shape=jax.ShapeDtypeStruct((B, N_RAYS, N_PTS), jnp.float32),
    )(base.reshape(1, N_PTS))

    xys = xys4.reshape(B, 2, N_RAYS).transpose(0, 2, 1)
    directions = dir4.reshape(B, 3, N_RAYS).transpose(0, 2, 1)
    origins = org4.reshape(B, 3, N_RAYS).transpose(0, 2, 1)
    return origins, directions, lengths, xys


# speedup vs baseline: 1.0578x; 1.0000x over previous
import functools

import jax
import jax.numpy as jnp
import numpy as np
from jax.experimental import pallas as pl
from jax.experimental.pallas import tpu as pltpu

B = 64
H = 512
W = 512
HW = H * W
N_RAYS = 1024
N_PTS = 64
MIN_DEPTH = 0.1
MAX_DEPTH = 8.0

_K_IDX = (507451445, 1853169794)
_K_STRAT = (1948878966, 4237131848)

_TINY = float(np.finfo(np.float32).tiny)

_NT = 2048
_NCHUNK = HW // _NT
_NGROUP = N_RAYS // 8


def _u32(x):
    return jnp.uint32(x)


def _threefry(x0, x1, k0, k1):
    ks0 = np.uint32(k0)
    ks1 = np.uint32(k1)
    ks2 = np.uint32(int(ks0) ^ int(ks1) ^ 0x1BD11BDA)
    ks = (ks0, ks1, ks2)
    rots = ((13, 15, 26, 6), (17, 29, 16, 24))
    for d in range(5):
        for r in rots[d % 2]:
            x0 = x0 + x1
            x1 = (x1 << _u32(r)) | (x1 >> _u32(32 - r))
            x1 = x1 ^ x0
        x0 = x0 + ks[(d + 1) % 3]
        x1 = x1 + np.uint32((int(ks[(d + 2) % 3]) + d + 1) & 0xFFFFFFFF)
    return x0, x1


def _bits_to_unit(bits):
    fb = (bits >> _u32(9)) | _u32(0x3F800000)
    return jax.lax.bitcast_convert_type(fb, jnp.float32) - jnp.float32(1.0)


def _race_body(mask_ref, idx_ref, invp_ref):
    b = pl.program_id(0)
    invp_ref[...] = jnp.float32(1.0) / jnp.maximum(mask_ref[0], jnp.float32(1e-12))

    x0_init = (jnp.uint32(b) >> _u32(4)) + _u32(_K_IDX[0])
    lo_base = ((jnp.uint32(b) & _u32(15)) * _u32(N_RAYS)) << _u32(18)

    tile_iota = (
        jax.lax.broadcasted_iota(jnp.uint32, (8, _NT), 0) << _u32(18)
    ) | jax.lax.broadcasted_iota(jnp.uint32, (8, _NT), 1)
    lane_i32 = jax.lax.broadcasted_iota(jnp.int32, (8, _NT), 1)

    def group_body(g, _):
        ray0 = g * 8
        grp_base = lo_base + (jnp.uint32(ray0) << _u32(18)) + _u32(_K_IDX[1])

        def chunk_body(c, carry):
            best, bidx = carry
            x1 = tile_iota + (grp_base + jnp.uint32(c) * _u32(_NT))
            o0, o1 = _threefry(x0_init, x1, _K_IDX[0], _K_IDX[1])
            u = jnp.maximum(_bits_to_unit(o0 ^ o1), jnp.float32(_TINY))
            invp = invp_ref[c, :]
            score = jnp.log(u) * jnp.broadcast_to(invp[None, :], (8, _NT))
            cur = lane_i32 + c * _NT
            take = score > best
            return (jnp.where(take, score, best), jnp.where(take, cur, bidx))

        init = (jnp.full((8, _NT), -jnp.inf, jnp.float32),
                jnp.zeros((8, _NT), jnp.int32))
        best, bidx = jax.lax.fori_loop(0, _NCHUNK, chunk_body, init)
        mx = jnp.max(best, axis=1, keepdims=True)
        win = jnp.min(jnp.where(best == mx, bidx, jnp.int32(2**30)), axis=1)
        idx_ref[0, g, :] = win
        return 0

    jax.lax.fori_loop(0, _NGROUP, group_body, 0)


def _post_body(idx_ref, r9_ref, t_ref, fl_ref, pp_ref, xys_ref, dir_ref, org_ref):
    idx = idx_ref[0, 0, :]
    wcol = idx & jnp.int32(W - 1)
    hrow = idx >> jnp.int32(9)
    c0 = jnp.float32(1.0 - 1.0 / W)
    step = jnp.float32(1.0 / 256.0)
    xf = c0 - wcol.astype(jnp.float32) * step
    yf = c0 - hrow.astype(jnp.float32) * step

    a, bb, c = r9_ref[0, 0, 0], r9_ref[0, 0, 1], r9_ref[0, 0, 2]
    d, e, f = r9_ref[0, 0, 3], r9_ref[0, 0, 4], r9_ref[0, 0, 5]
    g, h, i = r9_ref[0, 0, 6], r9_ref[0, 0, 7], r9_ref[0, 0, 8]
    det = a * (e * i - f * h) - bb * (d * i - f * g) + c * (d * h - e * g)
    inv_det = jnp.float32(1.0) / det
    i00 = (e * i - f * h) * inv_det
    i01 = (c * h - bb * i) * inv_det
    i02 = (bb * f - c * e) * inv_det
    i10 = (f * g - d * i) * inv_det
    i11 = (a * i - c * g) * inv_det
    i12 = (c * d - a * f) * inv_det
    i20 = (d * h - e * g) * inv_det
    i21 = (bb * g - a * h) * inv_det
    i22 = (a * e - bb * d) * inv_det

    px, py = pp_ref[0, 0, 0], pp_ref[0, 0, 1]
    fx, fy = fl_ref[0, 0, 0], fl_ref[0, 0, 1]
    dx = (xf - px) / fx
    dy = (yf - py) / fy

    d0 = dx * i00 + dy * i10 + i20
    d1 = dx * i01 + dy * i11 + i21
    d2 = dx * i02 + dy * i12 + i22
    nrm = jnp.sqrt(d0 * d0 + d1 * d1 + d2 * d2)
    dir_ref[0, 0, 0, :] = d0 / nrm
    dir_ref[0, 1, 0, :] = d1 / nrm
    dir_ref[0, 2, 0, :] = d2 / nrm

    t0, t1, t2 = t_ref[0, 0, 0], t_ref[0, 0, 1], t_ref[0, 0, 2]
    cc0 = -(t0 * i00 + t1 * i10 + t2 * i20)
    cc1 = -(t0 * i01 + t1 * i11 + t2 * i21)
    cc2 = -(t0 * i02 + t1 * i12 + t2 * i22)
    org_ref[0, 0, 0, :] = jnp.broadcast_to(cc0, (N_RAYS,))
    org_ref[0, 1, 0, :] = jnp.broadcast_to(cc1, (N_RAYS,))
    org_ref[0, 2, 0, :] = jnp.broadcast_to(cc2, (N_RAYS,))

    xys_ref[0, 0, 0, :] = xf
    xys_ref[0, 1, 0, :] = yf


def _lengths_body(base_ref, out_ref):
    b = pl.program_id(0)
    e = (
        jax.lax.broadcasted_iota(jnp.uint32, (N_RAYS, N_PTS), 0) * _u32(N_PTS)
        + jax.lax.broadcasted_iota(jnp.uint32, (N_RAYS, N_PTS), 1)
        + jnp.uint32(b) * _u32(N_RAYS * N_PTS)
        + _u32(_K_STRAT[1])
    )
    o0, o1 = _threefry(_u32(_K_STRAT[0]), e, _K_STRAT[0], _K_STRAT[1])
    u = _bits_to_unit(o0 ^ o1)
    delta = np.float32((MAX_DEPTH - MIN_DEPTH) / (N_PTS - 1))
    jit = (u - jnp.float32(0.5)) * delta
    out_ref[0] = jnp.broadcast_to(base_ref[0][None, :], (N_RAYS, N_PTS)) + jit


@functools.partial(jax.jit)
def kernel(mask, R, T, focal_length, principal_point):
    mask3 = mask.reshape(B, _NCHUNK, _NT)

    idx3 = pl.pallas_call(
        _race_body,
        grid=(B,),
        in_specs=[pl.BlockSpec((1, _NCHUNK, _NT), lambda b: (b, 0, 0))],
        out_specs=pl.BlockSpec((1, _NGROUP, 8), lambda b: (b, 0, 0)),
        out_shape=jax.ShapeDtypeStruct((B, _NGROUP, 8), jnp.int32),
        scratch_shapes=[pltpu.VMEM((_NCHUNK, _NT), jnp.float32)],
        compiler_params=pltpu.CompilerParams(
            dimension_semantics=("parallel",),
        ),
    )(mask3)

    idxf = idx3.reshape(B, 1, N_RAYS)
    r9 = R.reshape(B, 1, 9)

    smem = functools.partial(pl.BlockSpec, memory_space=pltpu.SMEM)
    xys4, dir4, org4 = pl.pallas_call(
        _post_body,
        grid=(B,),
        in_specs=[
            pl.BlockSpec((1, 1, N_RAYS), lambda b: (b, 0, 0)),
            smem((1, 1, 9), lambda b: (b, 0, 0)),
            smem((1, 1, 3), lambda b: (b, 0, 0)),
            smem((1, 1, 2), lambda b: (b, 0, 0)),
            smem((1, 1, 2), lambda b: (b, 0, 0)),
        ],
        out_specs=[
            pl.BlockSpec((1, 2, 1, N_RAYS), lambda b: (b, 0, 0, 0)),
            pl.BlockSpec((1, 3, 1, N_RAYS), lambda b: (b, 0, 0, 0)),
            pl.BlockSpec((1, 3, 1, N_RAYS), lambda b: (b, 0, 0, 0)),
        ],
        out_shape=[
            jax.ShapeDtypeStruct((B, 2, 1, N_RAYS), jnp.float32),
            jax.ShapeDtypeStruct((B, 3, 1, N_RAYS), jnp.float32),
            jax.ShapeDtypeStruct((B, 3, 1, N_RAYS), jnp.float32),
        ],
    )(idxf, r9, T.reshape(B, 1, 3), focal_length.reshape(B, 1, 2),
      principal_point.reshape(B, 1, 2))

    base = jnp.linspace(MIN_DEPTH, MAX_DEPTH, N_PTS, dtype=jnp.float32)
    lengths = pl.pallas_call(
        _lengths_body,
        grid=(B,),
        in_specs=[pl.BlockSpec((1, N_PTS), lambda b: (0, 0))],
        out_specs=pl.BlockSpec((1, N_RAYS, N_PTS), lambda b: (b, 0, 0)),
        out_shape=jax.ShapeDtypeStruct((B, N_RAYS, N_PTS), jnp.float32),
    )(base.reshape(1, N_PTS))

    xys = xys4.reshape(B, 2, N_RAYS).transpose(0, 2, 1)
    directions = dir4.reshape(B, 3, N_RAYS).transpose(0, 2, 1)
    origins = org4.reshape(B, 3, N_RAYS).transpose(0, 2, 1)
    return origins, directions, lengths, xys
